# baseline (speedup 1.0000x reference)
import jax
import jax.numpy as jnp
from jax.experimental import pallas as pl

_N_CODES = 20
_REPEAT = 10
_TOTAL = _N_CODES * _REPEAT


def _body(x_ref, w_ref, o_ref):
    w = w_ref[...]
    rows = jax.lax.broadcasted_iota(jnp.int32, (_TOTAL, _N_CODES), 0) // _REPEAT
    cols = jax.lax.broadcasted_iota(jnp.int32, (_TOTAL, _N_CODES), 1)
    onehot = (rows == cols).astype(jnp.float32)
    wexp = jax.lax.dot(onehot, w, precision=jax.lax.Precision.HIGHEST)
    o_ref[...] = x_ref[...] + wexp[None, :, :]


def kernel(X, W):
    B, T, D = X.shape
    BLK = 64
    return pl.pallas_call(
        _body,
        grid=(B // BLK,),
        in_specs=[
            pl.BlockSpec((BLK, T, D), lambda i: (i, 0, 0)),
            pl.BlockSpec((_N_CODES, D), lambda i: (0, 0)),
        ],
        out_specs=pl.BlockSpec((BLK, T, D), lambda i: (i, 0, 0)),
        out_shape=jax.ShapeDtypeStruct(X.shape, X.dtype),
    )(X, W)
